# baseline (speedup 1.0000x reference)
import jax
import jax.numpy as jnp
from jax import lax
from jax.experimental import pallas as pl
from jax.experimental.pallas import tpu as pltpu
from jax.experimental.pallas import tpu_sc as plsc

_USERS = 25000
_ITEMS = 25000
_N = _USERS + _ITEMS
_NPAD = 50048
_D = 64
_H = _D // 2
_NS = 16
_CHUNK = 512
_CPS = 98
_EPAD = _NS * _CPS * _CHUNK
_NSTR = _CHUNK // 128
_ROWS_PER_SUB = _NPAD // _NS
_BLK = 136
_NBLK = _ROWS_PER_SUB // _BLK


def _body(tbl0, src_hbm, dst_hbm, val_hbm, zeros_hbm, t1, t2, mean_hbm,
          rows_v, src_a, dst_a, vals_a, src_b, dst_b, vals_b, mo_v, acc,
          gsem, ssem, esem):
    c = lax.axis_index("c")
    s = lax.axis_index("s")
    coff = c * _NPAD

    def zero_acc():
        pltpu.sync_copy(zeros_hbm, acc.at[pl.ds(s * _ROWS_PER_SUB, _ROWS_PER_SUB)])

    def edge_slices(ci):
        e0 = (s * _CPS + ci) * _CHUNK
        return (src_hbm.at[pl.ds(e0, _CHUNK)],
                dst_hbm.at[pl.ds(e0, _CHUNK)],
                val_hbm.at[pl.ds(e0, _CHUNK)])

    def fire_edges(ci, buf):
        sv, dv, vv_ = buf
        for hs, vb in zip(edge_slices(ci), (sv, dv, vv_)):
            pltpu.async_copy(hs, vb, esem)

    def wait_edges(ci, buf):
        sv, dv, vv_ = buf
        for hs, vb in zip(edge_slices(ci), (sv, dv, vv_)):
            pltpu.make_async_copy(hs, vb, esem).wait()

    def process_chunk(src_tbl, buf):
        sv, dv, vv_ = buf

        @plsc.parallel_loop(0, _CHUNK, step=16)
        def _(j):
            sv[pl.ds(j, 16)] = sv[pl.ds(j, 16)] + coff

        pltpu.async_copy(src_tbl.at[sv], rows_v, gsem).wait()

        @plsc.parallel_loop(0, _CHUNK, step=16)
        def _(e0):
            vv = vv_[pl.ds(e0, 16)]
            for j in range(16):
                v = vv[j]
                rows_v[e0 + j, pl.ds(0, 16)] = (
                    rows_v[e0 + j, pl.ds(0, 16)] * v)
                rows_v[e0 + j, pl.ds(16, 16)] = (
                    rows_v[e0 + j, pl.ds(16, 16)] * v)

        pltpu.async_copy(rows_v, acc.at[dv], ssem, add=True).wait()

    def layer(src_tbl, out_tbl, buf_a, buf_b):
        zero_acc()
        plsc.subcore_barrier()

        fire_edges(0, buf_a)

        @pl.loop(0, _CPS // 2)
        def _(k):
            fire_edges(2 * k + 1, buf_b)
            wait_edges(2 * k, buf_a)
            process_chunk(src_tbl, buf_a)

            @pl.when(k < _CPS // 2 - 1)
            def _():
                fire_edges(2 * k + 2, buf_a)

            wait_edges(2 * k + 1, buf_b)
            process_chunk(src_tbl, buf_b)

        plsc.subcore_barrier()
        if out_tbl is not None:
            r0 = s * _ROWS_PER_SUB
            pltpu.sync_copy(acc.at[pl.ds(r0, _ROWS_PER_SUB)],
                            out_tbl.at[pl.ds(coff + r0, _ROWS_PER_SUB)])
            plsc.subcore_barrier()

    buf_a = (src_a, dst_a, vals_a)
    buf_b = (src_b, dst_b, vals_b)
    layer(tbl0, t1, buf_a, buf_b)
    layer(t1, t2, buf_a, buf_b)
    layer(t2, None, buf_a, buf_b)

    @pl.loop(0, _NBLK)
    def _(b):
        r0 = s * _ROWS_PER_SUB + b * _BLK
        g0 = coff + r0
        pltpu.sync_copy(tbl0.at[pl.ds(g0, _BLK)], rows_v.at[pl.ds(0, _BLK)])
        pltpu.sync_copy(t1.at[pl.ds(g0, _BLK)], rows_v.at[pl.ds(_BLK, _BLK)])
        pltpu.sync_copy(t2.at[pl.ds(g0, _BLK)],
                        rows_v.at[pl.ds(2 * _BLK, _BLK)])
        pltpu.sync_copy(acc.at[pl.ds(r0, _BLK)], mo_v)

        @pl.loop(0, _BLK)
        def _(i):
            for h in (0, 16):
                mo_v[i, pl.ds(h, 16)] = (
                    mo_v[i, pl.ds(h, 16)]
                    + rows_v[i, pl.ds(h, 16)]
                    + rows_v[_BLK + i, pl.ds(h, 16)]
                    + rows_v[2 * _BLK + i, pl.ds(h, 16)]) * 0.25
        pltpu.sync_copy(mo_v, mean_hbm.at[pl.ds(g0, _BLK)])


_TBL = jax.ShapeDtypeStruct((2 * _NPAD, _H), jnp.float32)


@jax.jit
def _propagate(tbl0, src_p, dst_p, val_p, zeros):
    f = pl.kernel(
        _body,
        out_type=(_TBL, _TBL, _TBL),
        mesh=plsc.VectorSubcoreMesh(core_axis_name="c", subcore_axis_name="s"),
        compiler_params=pltpu.CompilerParams(use_tc_tiling_on_sc=False),
        scratch_types=[
            pltpu.VMEM((_CHUNK, _H), jnp.float32),
            pltpu.VMEM((_CHUNK,), jnp.int32),
            pltpu.VMEM((_CHUNK,), jnp.int32),
            pltpu.VMEM((_CHUNK,), jnp.float32),
            pltpu.VMEM((_CHUNK,), jnp.int32),
            pltpu.VMEM((_CHUNK,), jnp.int32),
            pltpu.VMEM((_CHUNK,), jnp.float32),
            pltpu.VMEM((_BLK, _H), jnp.float32),
            pltpu.VMEM_SHARED((_NPAD, _H), jnp.float32),
            pltpu.SemaphoreType.DMA,
            pltpu.SemaphoreType.DMA,
            pltpu.SemaphoreType.DMA,
        ],
    )
    return f(tbl0, src_p, dst_p, val_p, zeros)


def kernel(edge_index, adj_values, user_emb, item_emb):
    ego = jnp.concatenate([user_emb, item_emb], axis=0)
    zpad = jnp.zeros((_NPAD - _N, _H), jnp.float32)
    tbl0 = jnp.concatenate([ego[:, :_H], zpad, ego[:, _H:], zpad], axis=0)
    src = edge_index[1].astype(jnp.int32)
    dst = edge_index[0].astype(jnp.int32)
    pad = _EPAD - src.shape[0]
    src_p = jnp.pad(src, (0, pad))
    dst_p = jnp.pad(dst, (0, pad))
    val_p = jnp.pad(adj_values, (0, pad))
    zeros = jnp.zeros((_ROWS_PER_SUB, _H), jnp.float32)
    _, _, mean = _propagate(tbl0, src_p, dst_p, val_p, zeros)
    out = jnp.concatenate([mean[:_N], mean[_NPAD:_NPAD + _N]], axis=1)
    return out[:_USERS], out[_USERS:]


# speedup vs baseline: 1.1507x; 1.1507x over previous
import jax
import jax.numpy as jnp
from jax import lax
from jax.experimental import pallas as pl
from jax.experimental.pallas import tpu as pltpu
from jax.experimental.pallas import tpu_sc as plsc

_USERS = 25000
_ITEMS = 25000
_N = _USERS + _ITEMS
_NPAD = 50048
_D = 64
_H = _D // 2
_NS = 16
_CHUNK = 512
_CPS = 98
_EPAD = _NS * _CPS * _CHUNK
_QROW = 256
_NSTR = _CHUNK // _QROW
_ROWS_PER_SUB = _NPAD // _NS
_BLK = 136
_NBLK = _ROWS_PER_SUB // _BLK


def _body(tbl0, src_hbm, dst_hbm, val_hbm, zeros_hbm, t1, t2, mean_hbm,
          rows_v, src_a, dst_a, vals_a, src_b, dst_b, vals_b, mo_v, acc,
          gsem, ssem, esem):
    c = lax.axis_index("c")
    s = lax.axis_index("s")
    coff = c * _NPAD

    def zero_acc():
        pltpu.sync_copy(zeros_hbm, acc.at[pl.ds(s * _ROWS_PER_SUB, _ROWS_PER_SUB)])

    def edge_slices(ci):
        row0 = (s * _CPS + ci) * _NSTR
        e0 = (s * _CPS + ci) * _CHUNK
        return (src_hbm.at[pl.ds(row0, _NSTR)],
                dst_hbm.at[pl.ds(row0, _NSTR)],
                val_hbm.at[pl.ds(e0, _CHUNK)])

    def fire_edges(ci, buf):
        sv, dv, vv_ = buf
        for hs, vb in zip(edge_slices(ci), (sv, dv, vv_)):
            pltpu.async_copy(hs, vb, esem)

    def wait_edges(ci, buf):
        sv, dv, vv_ = buf
        for hs, vb in zip(edge_slices(ci), (sv, dv, vv_)):
            pltpu.make_async_copy(hs, vb, esem).wait()

    def process_chunk(src_tbl, buf):
        sv, dv, vv_ = buf

        @plsc.parallel_loop(0, _NSTR)
        def _(r):
            for j in range(_QROW // 16):
                sv[r, pl.ds(j * 16, 16)] = sv[r, pl.ds(j * 16, 16)] + coff

        def fire_gather(g):
            return pltpu.async_copy(src_tbl.at[sv.at[g]],
                                    rows_v.at[pl.ds(g * _QROW, _QROW)], gsem)

        gd = {g: fire_gather(g) for g in range(min(2, _NSTR))}
        sd = []
        for q in range(_NSTR):
            gd[q].wait()
            if q + 2 < _NSTR:
                gd[q + 2] = fire_gather(q + 2)

            @plsc.parallel_loop(q * _QROW, (q + 1) * _QROW, step=16)
            def _(e0):
                vv = vv_[pl.ds(e0, 16)]
                for j in range(16):
                    v = vv[j]
                    rows_v[e0 + j, pl.ds(0, 16)] = (
                        rows_v[e0 + j, pl.ds(0, 16)] * v)
                    rows_v[e0 + j, pl.ds(16, 16)] = (
                        rows_v[e0 + j, pl.ds(16, 16)] * v)

            sd.append(pltpu.async_copy(rows_v.at[pl.ds(q * _QROW, _QROW)],
                                       acc.at[dv.at[q]], ssem,
                                       add=True))
        for d in sd:
            d.wait()

    def layer(src_tbl, out_tbl, buf_a, buf_b):
        zero_acc()
        plsc.subcore_barrier()

        fire_edges(0, buf_a)

        @pl.loop(0, _CPS // 2)
        def _(k):
            fire_edges(2 * k + 1, buf_b)
            wait_edges(2 * k, buf_a)
            process_chunk(src_tbl, buf_a)

            @pl.when(k < _CPS // 2 - 1)
            def _():
                fire_edges(2 * k + 2, buf_a)

            wait_edges(2 * k + 1, buf_b)
            process_chunk(src_tbl, buf_b)

        plsc.subcore_barrier()
        if out_tbl is not None:
            r0 = s * _ROWS_PER_SUB
            pltpu.sync_copy(acc.at[pl.ds(r0, _ROWS_PER_SUB)],
                            out_tbl.at[pl.ds(coff + r0, _ROWS_PER_SUB)])
            plsc.subcore_barrier()

    buf_a = (src_a, dst_a, vals_a)
    buf_b = (src_b, dst_b, vals_b)
    layer(tbl0, t1, buf_a, buf_b)
    layer(t1, t2, buf_a, buf_b)
    layer(t2, None, buf_a, buf_b)

    @pl.loop(0, _NBLK)
    def _(b):
        r0 = s * _ROWS_PER_SUB + b * _BLK
        g0 = coff + r0
        pltpu.sync_copy(tbl0.at[pl.ds(g0, _BLK)], rows_v.at[pl.ds(0, _BLK)])
        pltpu.sync_copy(t1.at[pl.ds(g0, _BLK)], rows_v.at[pl.ds(_BLK, _BLK)])
        pltpu.sync_copy(t2.at[pl.ds(g0, _BLK)],
                        rows_v.at[pl.ds(2 * _BLK, _BLK)])
        pltpu.sync_copy(acc.at[pl.ds(r0, _BLK)], mo_v)

        @pl.loop(0, _BLK)
        def _(i):
            for h in (0, 16):
                mo_v[i, pl.ds(h, 16)] = (
                    mo_v[i, pl.ds(h, 16)]
                    + rows_v[i, pl.ds(h, 16)]
                    + rows_v[_BLK + i, pl.ds(h, 16)]
                    + rows_v[2 * _BLK + i, pl.ds(h, 16)]) * 0.25
        pltpu.sync_copy(mo_v, mean_hbm.at[pl.ds(g0, _BLK)])


_TBL = jax.ShapeDtypeStruct((2 * _NPAD, _H), jnp.float32)


@jax.jit
def _propagate(tbl0, src_p, dst_p, val_p, zeros):
    f = pl.kernel(
        _body,
        out_type=(_TBL, _TBL, _TBL),
        mesh=plsc.VectorSubcoreMesh(core_axis_name="c", subcore_axis_name="s"),
        compiler_params=pltpu.CompilerParams(use_tc_tiling_on_sc=False),
        scratch_types=[
            pltpu.VMEM((_CHUNK, _H), jnp.float32),
            pltpu.VMEM((_NSTR, _QROW), jnp.int32),
            pltpu.VMEM((_NSTR, _QROW), jnp.int32),
            pltpu.VMEM((_CHUNK,), jnp.float32),
            pltpu.VMEM((_NSTR, _QROW), jnp.int32),
            pltpu.VMEM((_NSTR, _QROW), jnp.int32),
            pltpu.VMEM((_CHUNK,), jnp.float32),
            pltpu.VMEM((_BLK, _H), jnp.float32),
            pltpu.VMEM_SHARED((_NPAD, _H), jnp.float32),
            pltpu.SemaphoreType.DMA,
            pltpu.SemaphoreType.DMA,
            pltpu.SemaphoreType.DMA,
        ],
    )
    return f(tbl0, src_p, dst_p, val_p, zeros)


def kernel(edge_index, adj_values, user_emb, item_emb):
    ego = jnp.concatenate([user_emb, item_emb], axis=0)
    zpad = jnp.zeros((_NPAD - _N, _H), jnp.float32)
    tbl0 = jnp.concatenate([ego[:, :_H], zpad, ego[:, _H:], zpad], axis=0)
    src = edge_index[1].astype(jnp.int32)
    dst = edge_index[0].astype(jnp.int32)
    pad = _EPAD - src.shape[0]
    src_p = jnp.pad(src, (0, pad)).reshape(_EPAD // _QROW, _QROW)
    dst_p = jnp.pad(dst, (0, pad)).reshape(_EPAD // _QROW, _QROW)
    val_p = jnp.pad(adj_values, (0, pad))
    zeros = jnp.zeros((_ROWS_PER_SUB, _H), jnp.float32)
    _, _, mean = _propagate(tbl0, src_p, dst_p, val_p, zeros)
    out = jnp.concatenate([mean[:_N], mean[_NPAD:_NPAD + _N]], axis=1)
    return out[:_USERS], out[_USERS:]
